# baseline (speedup 1.0000x reference)
import functools

import jax
import jax.numpy as jnp
from jax import lax
from jax.experimental import pallas as pl
from jax.experimental.pallas import tpu as pltpu
from jax.experimental.pallas import tpu_sc as plsc

_EPS = 1e-5
_R = 512
_NSLICES = 4


def _sc_gather(slice_rows, H, per_w):
    mesh = plsc.VectorSubcoreMesh(core_axis_name="c", subcore_axis_name="s")
    NC = mesh.num_cores

    @functools.partial(
        pl.kernel,
        mesh=mesh,
        out_type=jax.ShapeDtypeStruct((slice_rows, H), jnp.float32),
        scratch_types=[
            pltpu.VMEM((per_w,), jnp.int32),
            pltpu.VMEM((per_w, H), jnp.float32),
            pltpu.SemaphoreType.DMA,
        ],
    )
    def gather_kernel(table_hbm, ids_hbm, out_hbm, idx_v, rows_v, sem):
        wid = lax.axis_index("s") * NC + lax.axis_index("c")
        base = wid * per_w
        pltpu.sync_copy(ids_hbm.at[pl.ds(base, per_w)], idx_v)
        pltpu.async_copy(table_hbm.at[idx_v], rows_v, sem).wait()
        pltpu.sync_copy(rows_v, out_hbm.at[pl.ds(base, per_w)])

    return gather_kernel


def _ln_first(frame_ref, pos_ref, w_ref, b_ref, out_ref):
    _ln_compute(frame_ref, pos_ref, w_ref, b_ref, out_ref)


def _ln_chain(buf_ref, frame_ref, pos_ref, w_ref, b_ref, out_ref):
    del buf_ref
    _ln_compute(frame_ref, pos_ref, w_ref, b_ref, out_ref)


def _ln_compute(frame_ref, pos_ref, w_ref, b_ref, out_ref):
    emb = frame_ref[...] + pos_ref[...]
    mean = jnp.mean(emb, axis=1, keepdims=True)
    cent = emb - mean
    var = jnp.mean(cent * cent, axis=1, keepdims=True)
    normed = cent * lax.rsqrt(var + _EPS)
    out_ref[...] = normed * w_ref[...] + b_ref[...]


def kernel(frame_feat, position_ids, pos_table, ln_weight, ln_bias):
    B, S, H = frame_feat.shape
    N = B * S
    slice_rows = N // _NSLICES
    blocks_per_slice = slice_rows // _R

    ids = position_ids.reshape(N).astype(jnp.int32)
    frame_r = frame_feat.reshape(N, H)
    w_r = ln_weight.reshape(1, H)
    b_r = ln_bias.reshape(1, H)

    gather = _sc_gather(slice_rows, H, slice_rows // 32)
    gathered = [
        gather(pos_table, lax.dynamic_slice_in_dim(ids, k * slice_rows, slice_rows))
        for k in range(_NSLICES)
    ]

    frame_spec = [
        pl.BlockSpec((_R, H), functools.partial(lambda k, i: (k * blocks_per_slice + i, 0), k))
        for k in range(_NSLICES)
    ]
    pos_spec = pl.BlockSpec((_R, H), lambda i: (i, 0))
    wb_spec = pl.BlockSpec((1, H), lambda i: (0, 0))
    out_shape = jax.ShapeDtypeStruct((N, H), jnp.float32)

    buf = pl.pallas_call(
        _ln_first,
        grid=(blocks_per_slice,),
        in_specs=[frame_spec[0], pos_spec, wb_spec, wb_spec],
        out_specs=pl.BlockSpec((_R, H), lambda i: (i, 0)),
        out_shape=out_shape,
    )(frame_r, gathered[0], w_r, b_r)

    for k in range(1, _NSLICES):
        buf = pl.pallas_call(
            _ln_chain,
            grid=(blocks_per_slice,),
            in_specs=[
                pl.BlockSpec(memory_space=pl.ANY),
                frame_spec[k],
                pos_spec,
                wb_spec,
                wb_spec,
            ],
            out_specs=pl.BlockSpec(
                (_R, H),
                functools.partial(lambda k, i: (k * blocks_per_slice + i, 0), k),
            ),
            out_shape=out_shape,
            input_output_aliases={0: 0},
        )(buf, frame_r, gathered[k], w_r, b_r)

    return buf.reshape(B, S, H)
